# baseline (speedup 1.0000x reference)
import functools
import numpy as np
import jax
import jax.numpy as jnp
from jax import lax
from jax.experimental import pallas as pl
from jax.experimental.pallas import tpu as pltpu
from jax.experimental.pallas import tpu_sc as plsc

N = 10000
NPAD = 10240
E = 160000
EPAD = 163840
B = 128
BP = 32
NC, NS = 2, 16
NW = NC * NS
EPT = EPAD // NW
NCHUNK = EPT // B
NCHUNKP = EPT // BP
RPS = NPAD // NS
DROW = 160
SROW2 = 256
ACC = 160
F = 32
HID = 120
INV_SQRT_H = float(1.0 / np.sqrt(120.0).astype(np.float32))
BM = 512
GRID = NPAD // BM

_f32 = jnp.float32
_i32 = jnp.int32

_MESH = dict(mesh=plsc.VectorSubcoreMesh(core_axis_name="c", subcore_axis_name="s"),
             compiler_params=pltpu.CompilerParams(needs_layout_passes=False,
                                                  use_tc_tiling_on_sc=False))


def _wid():
    return lax.axis_index("s") * NC + lax.axis_index("c")


def _rsqrt_sc(x):
    i = plsc.bitcast(x, _i32)
    i = 0x5F3759DF - lax.shift_right_logical(i, 1)
    y = plsc.bitcast(i, _f32)
    for _ in range(3):
        y = y * (1.5 - 0.5 * x * y * y)
    return y


def _geom_body(src, dst, ea, px, py, pz, ef_o,
               px_v, py_v, pz_v, si_v, di_v, ea_v, ef_v):
    base = _wid() * EPT
    pltpu.sync_copy(px, px_v)
    pltpu.sync_copy(py, py_v)
    pltpu.sync_copy(pz, pz_v)
    iota = lax.iota(_i32, 16)
    centers = np.linspace(0.0, 3.0, 16, dtype=np.float32)
    zcol = jnp.zeros((16,), _i32)

    def chunk(cc, _):
        off = pl.multiple_of(base + cc * B, B)
        pltpu.sync_copy(src.at[pl.ds(off, B)], si_v)
        pltpu.sync_copy(dst.at[pl.ds(off, B)], di_v)
        pltpu.sync_copy(ea.at[pl.ds(off, B)], ea_v)

        def group(g, _):
            g16 = pl.multiple_of(g * 16, 16)
            sv = si_v[pl.ds(g16, 16)]
            dv = di_v[pl.ds(g16, 16)]
            eav = ea_v[pl.ds(g16, 16)]
            rx = plsc.load_gather(px_v, [dv]) - plsc.load_gather(px_v, [sv])
            ry = plsc.load_gather(py_v, [dv]) - plsc.load_gather(py_v, [sv])
            rz = plsc.load_gather(pz_v, [dv]) - plsc.load_gather(pz_v, [sv])
            r2 = rx * rx + ry * ry + rz * rz + 1e-12
            rs = _rsqrt_sc(r2)
            r = r2 * rs
            rinv = 1.0 / (r + 1e-9)
            ux, uy, uz = rx * rinv, ry * rinv, rz * rinv
            rows = g16 + iota

            def put(col, vec):
                plsc.store_scatter(ef_v, [rows, zcol + col], vec)

            for k in range(16):
                t = r - centers[k]
                put(k, jnp.exp(t * t * -2.0))
            put(16, jnp.zeros((16,), _f32) + 1.0)
            put(17, ux)
            put(18, uy)
            put(19, uz)
            put(20, ux * uy)
            put(21, uy * uz)
            put(22, 0.5 * (3.0 * uz * uz - 1.0))
            put(23, ux * uz)
            put(24, 0.5 * (ux * ux - uy * uy))
            for k in range(6):
                put(25 + k, jnp.where(eav == k, 1.0, 0.0).astype(_f32))
            put(31, jnp.zeros((16,), _f32))
            return _

        lax.fori_loop(0, B // 16, group, None)
        pltpu.sync_copy(ef_v, ef_o.at[pl.ds(off, B)])
        return _

    lax.fori_loop(0, NCHUNK, chunk, None)


_geom = pl.kernel(
    _geom_body,
    out_type=jax.ShapeDtypeStruct((EPAD, F), _f32),
    scratch_types=[
        pltpu.VMEM((NPAD,), _f32), pltpu.VMEM((NPAD,), _f32), pltpu.VMEM((NPAD,), _f32),
        pltpu.VMEM((B,), _i32), pltpu.VMEM((B,), _i32), pltpu.VMEM((B,), _i32),
        pltpu.VMEM((B, F), _f32),
    ],
    **_MESH,
)


def _pass12_body(src, dst, drow, srow2, ef, acc_o,
                 si_v, di_v, drow_v, srow_v, ef_v, ab_v, agg_sh, sem, sem2):
    c = lax.axis_index("c")
    s = lax.axis_index("s")
    base = (s * NC + c) * EPT
    iota = lax.iota(_i32, 16)
    unit8 = jnp.where(iota == 8, 1.0, 0.0).astype(_f32)

    def zrow(r_, _):
        for t in range(ACC // 16):
            ab_v[r_, pl.ds(t * 16, 16)] = jnp.zeros((16,), _f32)
        return _

    lax.fori_loop(0, BP, zrow, None)
    for k in range(RPS // BP):
        pltpu.sync_copy(ab_v, agg_sh.at[pl.ds(s * RPS + k * BP, BP)])
    plsc.subcore_barrier()

    def chunk(cc, _):
        off = pl.multiple_of(base + cc * BP, BP)
        pltpu.sync_copy(src.at[pl.ds(off, BP)], si_v)
        pltpu.sync_copy(dst.at[pl.ds(off, BP)], di_v)
        cp1 = pltpu.async_copy(drow.at[di_v], drow_v, sem)
        cp2 = pltpu.async_copy(srow2.at[si_v], srow_v, sem2)
        pltpu.sync_copy(ef.at[pl.ds(off, BP)], ef_v)
        cp1.wait()
        cp2.wait()

        def group(g, _):
            g16 = pl.multiple_of(g * 16, 16)
            lvec = jnp.zeros((16,), _f32)
            for j in range(16):
                row = g16 + j
                acc = drow_v[row, pl.ds(0, 16)] * srow_v[row, pl.ds(0, 16)]
                for kk in range(1, 8):
                    acc = acc + drow_v[row, pl.ds(kk * 16, 16)] * srow_v[row, pl.ds(kk * 16, 16)]
                for kk in range(2):
                    acc = acc + drow_v[row, pl.ds(128 + kk * 16, 16)] * ef_v[row, pl.ds(kk * 16, 16)]
                sc = jnp.sum(acc)
                lvec = jnp.where(iota == j, sc, lvec)
            evec = jnp.exp(lvec * INV_SQRT_H)
            for j in range(16):
                row = g16 + j
                ee = jnp.sum(jnp.where(iota == j, evec, 0.0))
                for t in range(7):
                    ab_v[row, pl.ds(t * 16, 16)] = srow_v[row, pl.ds(128 + t * 16, 16)] * ee
                ab_v[row, pl.ds(112, 16)] = (srow_v[row, pl.ds(240, 16)] + unit8) * ee
                for t in range(2):
                    ab_v[row, pl.ds(128 + t * 16, 16)] = ef_v[row, pl.ds(t * 16, 16)] * ee
            return _

        lax.fori_loop(0, BP // 16, group, None)
        pltpu.sync_copy(ab_v, agg_sh.at[di_v], add=True)
        return _

    lax.fori_loop(0, NCHUNKP, chunk, None)
    plsc.subcore_barrier()
    nbase = s * RPS
    pltpu.sync_copy(agg_sh.at[pl.ds(nbase, RPS)], acc_o.at[c, pl.ds(nbase, RPS)])


_pass12 = pl.kernel(
    _pass12_body,
    out_type=jax.ShapeDtypeStruct((NC, NPAD, ACC), _f32),
    scratch_types=[
        pltpu.VMEM((BP,), _i32), pltpu.VMEM((BP,), _i32),
        pltpu.VMEM((BP, DROW), _f32), pltpu.VMEM((BP, SROW2), _f32),
        pltpu.VMEM((BP, F), _f32), pltpu.VMEM((BP, ACC), _f32),
        pltpu.VMEM_SHARED((NPAD, ACC), _f32),
        pltpu.SemaphoreType.DMA, pltpu.SemaphoreType.DMA,
    ],
    **_MESH,
)


def _full_spec(shape):
    nd = len(shape)
    return pl.BlockSpec(shape, lambda i, _nd=nd: (0,) * _nd)


def _rows_spec(shape):
    nd = len(shape)
    return pl.BlockSpec((BM,) + shape[1:], lambda i, _nd=nd: (i,) + (0,) * (_nd - 1))


def _embed_body(x0, x1, ei, eih, eat, eath, h_o, na_o):
    i16 = lax.broadcasted_iota(_i32, (1, 16), 1)
    oh0 = (x0[...] == i16).astype(_f32)
    oh1 = (x1[...] == i16).astype(_f32)
    h_o[...] = jnp.concatenate(
        [jnp.dot(oh0, ei[...], preferred_element_type=_f32),
         jnp.dot(oh1, eih[...], preferred_element_type=_f32)], axis=1)
    na_o[...] = jnp.concatenate(
        [jnp.dot(oh0, eat[...], preferred_element_type=_f32),
         jnp.dot(oh1, eath[...], preferred_element_type=_f32)], axis=1)


def _embed(x0, x1, ei, eih, eat, eath):
    return pl.pallas_call(
        _embed_body,
        grid=(GRID,),
        in_specs=[_rows_spec((NPAD, 1)), _rows_spec((NPAD, 1)),
                  _full_spec((16, 32)), _full_spec((16, 32)),
                  _full_spec((16, 32)), _full_spec((16, 32))],
        out_specs=[_rows_spec((NPAD, 64)), _rows_spec((NPAD, 64))],
        out_shape=[jax.ShapeDtypeStruct((NPAD, 64), _f32),
                   jax.ShapeDtypeStruct((NPAD, 64), _f32)],
    )(x0, x1, ei, eih, eat, eath)


def _pre_body(h, na, wq, wkn, wvn, wkft, drow_o, srow_o, *, pad):
    hn = jnp.concatenate([h[...], na[...]], axis=1)
    if pad:
        hn = jnp.concatenate([hn, jnp.zeros((BM, pad), _f32)], axis=1)
    q = jnp.dot(hn, wq[...], preferred_element_type=_f32)
    nk = jnp.dot(hn, wkn[...], preferred_element_type=_f32)
    nv = jnp.dot(hn, wvn[...], preferred_element_type=_f32)
    qe = jnp.dot(q, wkft[...], preferred_element_type=_f32)
    drow_o[...] = jnp.concatenate([q, qe], axis=1)
    srow_o[...] = jnp.concatenate([nk, nv], axis=1)


def _pre(h, na, wq, wkn, wvn, wkft, d_in, pad):
    phn = d_in + 64 + pad
    return pl.pallas_call(
        functools.partial(_pre_body, pad=pad),
        grid=(GRID,),
        in_specs=[_rows_spec((NPAD, d_in)), _rows_spec((NPAD, 64)),
                  _full_spec((phn, 128)), _full_spec((phn, 128)),
                  _full_spec((phn, 128)), _full_spec((128, 32))],
        out_specs=[_rows_spec((NPAD, DROW)), _rows_spec((NPAD, SROW2))],
        out_shape=[jax.ShapeDtypeStruct((NPAD, DROW), _f32),
                   jax.ShapeDtypeStruct((NPAD, SROW2), _f32)],
    )(h, na, wq, wkn, wvn, wkft)


def _ln_tc(v, g, b):
    mu = jnp.mean(v, axis=1, keepdims=True)
    var = jnp.mean((v - mu) * (v - mu), axis=1, keepdims=True)
    return (v - mu) / jnp.sqrt(var + 1e-5) * g + b


def _post_body(acc3, h, na, wvf, wo, g, b, w1, w2, fg, fb, ho,
               *, d_out, residual, has_ff, pad):
    accs = acc3[0] + acc3[1]
    zinv = 1.0 / (accs[:, 120:121] + 1e-9)
    aggv = accs[:, :d_out]
    aggf = accs[:, 128:160]
    a = (aggv + jnp.dot(aggf, wvf[...], preferred_element_type=_f32)) * zinv
    out = jnp.dot(a, wo[...], preferred_element_type=_f32)
    if residual:
        out = out + h[...]
    hh = _ln_tc(out, g[...], b[...])
    if has_ff:
        hn = jnp.concatenate([hh, na[...]], axis=1)
        if pad:
            hn = jnp.concatenate([hn, jnp.zeros((BM, pad), _f32)], axis=1)
        f = jax.nn.gelu(jnp.dot(hn, w1[...], preferred_element_type=_f32))
        f = jnp.dot(f, w2[...], preferred_element_type=_f32) + hh
        hh = _ln_tc(f, fg[...], fb[...])
    ho[...] = hh


def _post(acc3, h, na, wvf, wo, g, b, w1, w2, fg, fb, d_in, d_out,
          residual, has_ff, pad):
    phn = d_out + 64 + pad
    return pl.pallas_call(
        functools.partial(_post_body, d_out=d_out, residual=residual,
                          has_ff=has_ff, pad=pad),
        grid=(GRID,),
        in_specs=[pl.BlockSpec((NC, BM, ACC), lambda i: (0, i, 0)),
                  _rows_spec((NPAD, d_in)), _rows_spec((NPAD, 64)),
                  _full_spec((F, d_out)), _full_spec((d_out, d_out)),
                  _full_spec((1, d_out)), _full_spec((1, d_out)),
                  _full_spec((phn, 240)), _full_spec((240, 120)),
                  _full_spec((1, 120)), _full_spec((1, 120))],
        out_specs=_rows_spec((NPAD, d_out)),
        out_shape=jax.ShapeDtypeStruct((NPAD, d_out), _f32),
    )(acc3, h, na, wvf, wo, g, b, w1, w2, fg, fb)


def _head_body(h, w1, b1, w2, b2, o):
    u = jnp.dot(h[...], w1[...], preferred_element_type=_f32) + b1[...]
    t = jnp.where(u > 0, u, jnp.exp(u) - 1.0)
    o[...] = jnp.dot(t, w2[...], preferred_element_type=_f32) + b2[...]


def _head(h, w1, b1, w2, b2):
    return pl.pallas_call(
        _head_body,
        grid=(GRID,),
        in_specs=[_rows_spec((NPAD, 32)), _full_spec((32, 96)), _full_spec((1, 96)),
                  _full_spec((96, 128)), _full_spec((1, 128))],
        out_specs=_rows_spec((NPAD, 128)),
        out_shape=jax.ShapeDtypeStruct((NPAD, 128), _f32),
    )(h, w1, b1, w2, b2)


def _pad_rows(a, rows):
    return jnp.concatenate(
        [a, jnp.zeros((rows - a.shape[0],) + a.shape[1:], a.dtype)], axis=0)


def _pad_cols(a, cols):
    return jnp.concatenate(
        [a, jnp.zeros(a.shape[:-1] + (cols - a.shape[-1],), a.dtype)], axis=-1)


def kernel(x, pos, edge_index, edge_attr, params):
    p = params
    perm = jnp.argsort(edge_index[1])
    src = _pad_rows(edge_index[0][perm].astype(_i32), EPAD)
    dst = jnp.concatenate([edge_index[1][perm].astype(_i32),
                           jnp.full((EPAD - E,), N, _i32)])
    ea = _pad_rows(edge_attr[perm].astype(_i32), EPAD)
    posp = _pad_rows(pos.astype(_f32), NPAD)
    px, py, pz = posp[:, 0], posp[:, 1], posp[:, 2]
    x0 = _pad_rows(x[:, 0:1].astype(_i32), NPAD)
    x1 = _pad_rows(x[:, 1:2].astype(_i32), NPAD)

    ef = _geom(src, dst, ea, px, py, pz)
    h, na = _embed(x0, x1,
                   _pad_rows(p['emb_in'], 16), _pad_rows(p['emb_in_h'], 16),
                   _pad_rows(p['emb_attr'], 16), _pad_rows(p['emb_attr_h'], 16))

    d_in = 64
    for i in range(6):
        lp = p['layers'][i]
        d_out = 120 if i < 5 else 32
        nb = d_in + 64
        pad = (-nb) % 16
        Wk, Wv = lp['Wk'], lp['Wv']
        wkf = jnp.concatenate([Wk[nb:nb + 16], Wk[nb + 48:],
                               p['emb_edge'] @ Wk[nb + 16:nb + 48],
                               jnp.zeros((1, HID), _f32)], axis=0)
        wvf = jnp.concatenate([Wv[nb:nb + 16], Wv[nb + 48:],
                               p['emb_edge'] @ Wv[nb + 16:nb + 48],
                               jnp.zeros((1, d_out), _f32)], axis=0)
        wq = _pad_cols(_pad_rows(lp['Wq'], nb + pad), 128)
        wkn = _pad_cols(_pad_rows(Wk[:nb], nb + pad), 128)
        wvn = _pad_cols(_pad_rows(Wv[:nb], nb + pad), 128)
        wkft = _pad_rows(wkf.T, 128)

        drow, srow2 = _pre(h, na, wq, wkn, wvn, wkft, d_in, pad)
        acc3 = _pass12(src, dst, drow, srow2, ef)

        has_ff = i < 5
        if has_ff:
            fp = p['ff'][i]
            w1 = _pad_rows(fp['W1'], d_out + 64 + ((-(d_out + 64)) % 16))
            w2, fg, fb = fp['W2'], fp['ln_g'][None], fp['ln_b'][None]
        else:
            w1 = jnp.zeros((d_out + 64 + ((-(d_out + 64)) % 16), 240), _f32)
            w2 = jnp.zeros((240, 120), _f32)
            fg = jnp.zeros((1, 120), _f32)
            fb = jnp.zeros((1, 120), _f32)
        h = _post(acc3, h, na, wvf, lp['Wo'], lp['ln_g'][None], lp['ln_b'][None],
                  w1, w2, fg, fb, d_in, d_out,
                  residual=(d_out == d_in), has_ff=has_ff,
                  pad=(-(d_out + 64)) % 16)
        d_in = d_out

    o = _head(h, p['out_W1'], p['out_b1'][None],
              _pad_cols(p['out_W2'], 128), _pad_cols(p['out_b2'][None], 128))
    return o[:N, 0:1]


# speedup vs baseline: 1.2534x; 1.2534x over previous
import functools
import numpy as np
import jax
import jax.numpy as jnp
from jax import lax
from jax.experimental import pallas as pl
from jax.experimental.pallas import tpu as pltpu
from jax.experimental.pallas import tpu_sc as plsc

N = 10000
NPAD = 10240
E = 160000
EPAD = 163840
B = 128
BP = 32
HC = 16
NC, NS = 2, 16
NW = NC * NS
EPT = EPAD // NW
NH = EPT // HC
NCHUNK = EPT // B
NCHUNKP = EPT // BP
RPS = NPAD // NS
DROW = 160
SROW2 = 256
ACC = 160
F = 32
HID = 120
INV_SQRT_H = float(1.0 / np.sqrt(120.0).astype(np.float32))
BM = 512
GRID = NPAD // BM

_f32 = jnp.float32
_i32 = jnp.int32

_MESH = dict(mesh=plsc.VectorSubcoreMesh(core_axis_name="c", subcore_axis_name="s"),
             compiler_params=pltpu.CompilerParams(needs_layout_passes=False,
                                                  use_tc_tiling_on_sc=False))


def _wid():
    return lax.axis_index("s") * NC + lax.axis_index("c")


def _rsqrt_sc(x):
    i = plsc.bitcast(x, _i32)
    i = 0x5F3759DF - lax.shift_right_logical(i, 1)
    y = plsc.bitcast(i, _f32)
    for _ in range(3):
        y = y * (1.5 - 0.5 * x * y * y)
    return y


def _geom_body(src, dst, ea, px, py, pz, ef_o,
               px_v, py_v, pz_v, si_v, di_v, ea_v, ef_v):
    base = _wid() * EPT
    pltpu.sync_copy(px, px_v)
    pltpu.sync_copy(py, py_v)
    pltpu.sync_copy(pz, pz_v)
    iota = lax.iota(_i32, 16)
    centers = np.linspace(0.0, 3.0, 16, dtype=np.float32)
    zcol = jnp.zeros((16,), _i32)

    def chunk(cc, _):
        off = pl.multiple_of(base + cc * B, B)
        pltpu.sync_copy(src.at[pl.ds(off, B)], si_v)
        pltpu.sync_copy(dst.at[pl.ds(off, B)], di_v)
        pltpu.sync_copy(ea.at[pl.ds(off, B)], ea_v)

        def group(g, _):
            g16 = pl.multiple_of(g * 16, 16)
            sv = si_v[pl.ds(g16, 16)]
            dv = di_v[pl.ds(g16, 16)]
            eav = ea_v[pl.ds(g16, 16)]
            rx = plsc.load_gather(px_v, [dv]) - plsc.load_gather(px_v, [sv])
            ry = plsc.load_gather(py_v, [dv]) - plsc.load_gather(py_v, [sv])
            rz = plsc.load_gather(pz_v, [dv]) - plsc.load_gather(pz_v, [sv])
            r2 = rx * rx + ry * ry + rz * rz + 1e-12
            rs = _rsqrt_sc(r2)
            r = r2 * rs
            rinv = 1.0 / (r + 1e-9)
            ux, uy, uz = rx * rinv, ry * rinv, rz * rinv
            rows = g16 + iota

            def put(col, vec):
                plsc.store_scatter(ef_v, [rows, zcol + col], vec)

            for k in range(16):
                t = r - centers[k]
                put(k, jnp.exp(t * t * -2.0))
            put(16, jnp.zeros((16,), _f32) + 1.0)
            put(17, ux)
            put(18, uy)
            put(19, uz)
            put(20, ux * uy)
            put(21, uy * uz)
            put(22, 0.5 * (3.0 * uz * uz - 1.0))
            put(23, ux * uz)
            put(24, 0.5 * (ux * ux - uy * uy))
            for k in range(6):
                put(25 + k, jnp.where(eav == k, 1.0, 0.0).astype(_f32))
            put(31, jnp.zeros((16,), _f32))
            return _

        lax.fori_loop(0, B // 16, group, None)
        pltpu.sync_copy(ef_v, ef_o.at[pl.ds(off, B)])
        return _

    lax.fori_loop(0, NCHUNK, chunk, None)


_geom = pl.kernel(
    _geom_body,
    out_type=jax.ShapeDtypeStruct((EPAD, F), _f32),
    scratch_types=[
        pltpu.VMEM((NPAD,), _f32), pltpu.VMEM((NPAD,), _f32), pltpu.VMEM((NPAD,), _f32),
        pltpu.VMEM((B,), _i32), pltpu.VMEM((B,), _i32), pltpu.VMEM((B,), _i32),
        pltpu.VMEM((B, F), _f32),
    ],
    **_MESH,
)


def _pass12_body(src, dst, drow, srow2, ef, acc_o,
                 si_a, si_b, di_a, di_b, drow_a, drow_b, srow_a, srow_b,
                 ef_v, ab_v, agg_sh,
                 semd_a, semd_b, sems_a, sems_b):
    c = lax.axis_index("c")
    s = lax.axis_index("s")
    base = (s * NC + c) * EPT
    iota = lax.iota(_i32, 16)
    unit8 = jnp.where(iota == 8, 1.0, 0.0).astype(_f32)
    si = (si_a, si_b)
    di = (di_a, di_b)
    drow_v = (drow_a, drow_b)
    srow_v = (srow_a, srow_b)
    semd = (semd_a, semd_b)
    sems = (sems_a, sems_b)

    def zrow(r_, _):
        for t in range(ACC // 16):
            ab_v[r_, pl.ds(t * 16, 16)] = jnp.zeros((16,), _f32)
        return _

    lax.fori_loop(0, HC, zrow, None)
    for k in range(RPS // HC):
        pltpu.sync_copy(ab_v, agg_sh.at[pl.ds(s * RPS + k * HC, HC)])
    plsc.subcore_barrier()

    for p in range(2):
        offp = pl.multiple_of(base + p * HC, HC)
        pltpu.sync_copy(src.at[pl.ds(offp, HC)], si[p])
        pltpu.sync_copy(dst.at[pl.ds(offp, HC)], di[p])
        pltpu.async_copy(drow.at[di[p]], drow_v[p], semd[p])
        pltpu.async_copy(srow2.at[si[p]], srow_v[p], sems[p])

    def pair(g, _):
        for p in range(2):
            hc = 2 * g + p
            off = pl.multiple_of(base + hc * HC, HC)
            pltpu.make_async_copy(drow.at[pl.ds(0, HC)], drow_v[p], semd[p]).wait()
            pltpu.make_async_copy(srow2.at[pl.ds(0, HC)], srow_v[p], sems[p]).wait()
            pltpu.sync_copy(ef.at[pl.ds(off, HC)], ef_v)
            dv = drow_v[p]
            sv = srow_v[p]
            lvec = jnp.zeros((16,), _f32)
            for j in range(16):
                acc = dv[j, pl.ds(0, 16)] * sv[j, pl.ds(0, 16)]
                for kk in range(1, 8):
                    acc = acc + dv[j, pl.ds(kk * 16, 16)] * sv[j, pl.ds(kk * 16, 16)]
                for kk in range(2):
                    acc = acc + dv[j, pl.ds(128 + kk * 16, 16)] * ef_v[j, pl.ds(kk * 16, 16)]
                sc = jnp.sum(acc)
                lvec = jnp.where(iota == j, sc, lvec)
            evec = jnp.exp(lvec * INV_SQRT_H)
            for j in range(16):
                ee = jnp.sum(jnp.where(iota == j, evec, 0.0))
                for t in range(7):
                    ab_v[j, pl.ds(t * 16, 16)] = sv[j, pl.ds(128 + t * 16, 16)] * ee
                ab_v[j, pl.ds(112, 16)] = (sv[j, pl.ds(240, 16)] + unit8) * ee
                for t in range(2):
                    ab_v[j, pl.ds(128 + t * 16, 16)] = ef_v[j, pl.ds(t * 16, 16)] * ee
            pltpu.sync_copy(ab_v, agg_sh.at[di[p]], add=True)
            nxt = jnp.minimum(hc + 2, NH - 1)
            off2 = pl.multiple_of(base + nxt * HC, HC)
            pltpu.sync_copy(src.at[pl.ds(off2, HC)], si[p])
            pltpu.sync_copy(dst.at[pl.ds(off2, HC)], di[p])
            pltpu.async_copy(drow.at[di[p]], drow_v[p], semd[p])
            pltpu.async_copy(srow2.at[si[p]], srow_v[p], sems[p])
        return _

    lax.fori_loop(0, NH // 2, pair, None)
    for p in range(2):
        pltpu.make_async_copy(drow.at[pl.ds(0, HC)], drow_v[p], semd[p]).wait()
        pltpu.make_async_copy(srow2.at[pl.ds(0, HC)], srow_v[p], sems[p]).wait()
    plsc.subcore_barrier()
    nbase = s * RPS
    pltpu.sync_copy(agg_sh.at[pl.ds(nbase, RPS)], acc_o.at[c, pl.ds(nbase, RPS)])


_pass12 = pl.kernel(
    _pass12_body,
    out_type=jax.ShapeDtypeStruct((NC, NPAD, ACC), _f32),
    scratch_types=[
        pltpu.VMEM((HC,), _i32), pltpu.VMEM((HC,), _i32),
        pltpu.VMEM((HC,), _i32), pltpu.VMEM((HC,), _i32),
        pltpu.VMEM((HC, DROW), _f32), pltpu.VMEM((HC, DROW), _f32),
        pltpu.VMEM((HC, SROW2), _f32), pltpu.VMEM((HC, SROW2), _f32),
        pltpu.VMEM((HC, F), _f32), pltpu.VMEM((HC, ACC), _f32),
        pltpu.VMEM_SHARED((NPAD, ACC), _f32),
        pltpu.SemaphoreType.DMA, pltpu.SemaphoreType.DMA,
        pltpu.SemaphoreType.DMA, pltpu.SemaphoreType.DMA,
    ],
    **_MESH,
)


def _full_spec(shape):
    nd = len(shape)
    return pl.BlockSpec(shape, lambda i, _nd=nd: (0,) * _nd)


def _rows_spec(shape):
    nd = len(shape)
    return pl.BlockSpec((BM,) + shape[1:], lambda i, _nd=nd: (i,) + (0,) * (_nd - 1))


def _embed_body(x0, x1, ei, eih, eat, eath, h_o, na_o):
    i16 = lax.broadcasted_iota(_i32, (1, 16), 1)
    oh0 = (x0[...] == i16).astype(_f32)
    oh1 = (x1[...] == i16).astype(_f32)
    h_o[...] = jnp.concatenate(
        [jnp.dot(oh0, ei[...], preferred_element_type=_f32),
         jnp.dot(oh1, eih[...], preferred_element_type=_f32)], axis=1)
    na_o[...] = jnp.concatenate(
        [jnp.dot(oh0, eat[...], preferred_element_type=_f32),
         jnp.dot(oh1, eath[...], preferred_element_type=_f32)], axis=1)


def _embed(x0, x1, ei, eih, eat, eath):
    return pl.pallas_call(
        _embed_body,
        grid=(GRID,),
        in_specs=[_rows_spec((NPAD, 1)), _rows_spec((NPAD, 1)),
                  _full_spec((16, 32)), _full_spec((16, 32)),
                  _full_spec((16, 32)), _full_spec((16, 32))],
        out_specs=[_rows_spec((NPAD, 64)), _rows_spec((NPAD, 64))],
        out_shape=[jax.ShapeDtypeStruct((NPAD, 64), _f32),
                   jax.ShapeDtypeStruct((NPAD, 64), _f32)],
    )(x0, x1, ei, eih, eat, eath)


def _pre_body(h, na, wq, wkn, wvn, wkft, drow_o, srow_o, *, pad):
    hn = jnp.concatenate([h[...], na[...]], axis=1)
    if pad:
        hn = jnp.concatenate([hn, jnp.zeros((BM, pad), _f32)], axis=1)
    q = jnp.dot(hn, wq[...], preferred_element_type=_f32)
    nk = jnp.dot(hn, wkn[...], preferred_element_type=_f32)
    nv = jnp.dot(hn, wvn[...], preferred_element_type=_f32)
    qe = jnp.dot(q, wkft[...], preferred_element_type=_f32)
    drow_o[...] = jnp.concatenate([q, qe], axis=1)
    srow_o[...] = jnp.concatenate([nk, nv], axis=1)


def _pre(h, na, wq, wkn, wvn, wkft, d_in, pad):
    phn = d_in + 64 + pad
    return pl.pallas_call(
        functools.partial(_pre_body, pad=pad),
        grid=(GRID,),
        in_specs=[_rows_spec((NPAD, d_in)), _rows_spec((NPAD, 64)),
                  _full_spec((phn, 128)), _full_spec((phn, 128)),
                  _full_spec((phn, 128)), _full_spec((128, 32))],
        out_specs=[_rows_spec((NPAD, DROW)), _rows_spec((NPAD, SROW2))],
        out_shape=[jax.ShapeDtypeStruct((NPAD, DROW), _f32),
                   jax.ShapeDtypeStruct((NPAD, SROW2), _f32)],
    )(h, na, wq, wkn, wvn, wkft)


def _ln_tc(v, g, b):
    mu = jnp.mean(v, axis=1, keepdims=True)
    var = jnp.mean((v - mu) * (v - mu), axis=1, keepdims=True)
    return (v - mu) / jnp.sqrt(var + 1e-5) * g + b


def _post_body(acc3, h, na, wvf, wo, g, b, w1, w2, fg, fb, ho,
               *, d_out, residual, has_ff, pad):
    accs = acc3[0] + acc3[1]
    zinv = 1.0 / (accs[:, 120:121] + 1e-9)
    aggv = accs[:, :d_out]
    aggf = accs[:, 128:160]
    a = (aggv + jnp.dot(aggf, wvf[...], preferred_element_type=_f32)) * zinv
    out = jnp.dot(a, wo[...], preferred_element_type=_f32)
    if residual:
        out = out + h[...]
    hh = _ln_tc(out, g[...], b[...])
    if has_ff:
        hn = jnp.concatenate([hh, na[...]], axis=1)
        if pad:
            hn = jnp.concatenate([hn, jnp.zeros((BM, pad), _f32)], axis=1)
        f = jax.nn.gelu(jnp.dot(hn, w1[...], preferred_element_type=_f32))
        f = jnp.dot(f, w2[...], preferred_element_type=_f32) + hh
        hh = _ln_tc(f, fg[...], fb[...])
    ho[...] = hh


def _post(acc3, h, na, wvf, wo, g, b, w1, w2, fg, fb, d_in, d_out,
          residual, has_ff, pad):
    phn = d_out + 64 + pad
    return pl.pallas_call(
        functools.partial(_post_body, d_out=d_out, residual=residual,
                          has_ff=has_ff, pad=pad),
        grid=(GRID,),
        in_specs=[pl.BlockSpec((NC, BM, ACC), lambda i: (0, i, 0)),
                  _rows_spec((NPAD, d_in)), _rows_spec((NPAD, 64)),
                  _full_spec((F, d_out)), _full_spec((d_out, d_out)),
                  _full_spec((1, d_out)), _full_spec((1, d_out)),
                  _full_spec((phn, 240)), _full_spec((240, 120)),
                  _full_spec((1, 120)), _full_spec((1, 120))],
        out_specs=_rows_spec((NPAD, d_out)),
        out_shape=jax.ShapeDtypeStruct((NPAD, d_out), _f32),
    )(acc3, h, na, wvf, wo, g, b, w1, w2, fg, fb)


def _head_body(h, w1, b1, w2, b2, o):
    u = jnp.dot(h[...], w1[...], preferred_element_type=_f32) + b1[...]
    t = jnp.where(u > 0, u, jnp.exp(u) - 1.0)
    o[...] = jnp.dot(t, w2[...], preferred_element_type=_f32) + b2[...]


def _head(h, w1, b1, w2, b2):
    return pl.pallas_call(
        _head_body,
        grid=(GRID,),
        in_specs=[_rows_spec((NPAD, 32)), _full_spec((32, 96)), _full_spec((1, 96)),
                  _full_spec((96, 128)), _full_spec((1, 128))],
        out_specs=_rows_spec((NPAD, 128)),
        out_shape=jax.ShapeDtypeStruct((NPAD, 128), _f32),
    )(h, w1, b1, w2, b2)


def _pad_rows(a, rows):
    return jnp.concatenate(
        [a, jnp.zeros((rows - a.shape[0],) + a.shape[1:], a.dtype)], axis=0)


def _pad_cols(a, cols):
    return jnp.concatenate(
        [a, jnp.zeros(a.shape[:-1] + (cols - a.shape[-1],), a.dtype)], axis=-1)


def kernel(x, pos, edge_index, edge_attr, params):
    p = params
    src = _pad_rows(edge_index[0].astype(_i32), EPAD)
    dst = jnp.concatenate([edge_index[1].astype(_i32),
                           jnp.full((EPAD - E,), N, _i32)])
    ea = _pad_rows(edge_attr.astype(_i32), EPAD)
    posp = _pad_rows(pos.astype(_f32), NPAD)
    px, py, pz = posp[:, 0], posp[:, 1], posp[:, 2]
    x0 = _pad_rows(x[:, 0:1].astype(_i32), NPAD)
    x1 = _pad_rows(x[:, 1:2].astype(_i32), NPAD)

    ef = _geom(src, dst, ea, px, py, pz)
    h, na = _embed(x0, x1,
                   _pad_rows(p['emb_in'], 16), _pad_rows(p['emb_in_h'], 16),
                   _pad_rows(p['emb_attr'], 16), _pad_rows(p['emb_attr_h'], 16))

    d_in = 64
    for i in range(6):
        lp = p['layers'][i]
        d_out = 120 if i < 5 else 32
        nb = d_in + 64
        pad = (-nb) % 16
        Wk, Wv = lp['Wk'], lp['Wv']
        wkf = jnp.concatenate([Wk[nb:nb + 16], Wk[nb + 48:],
                               p['emb_edge'] @ Wk[nb + 16:nb + 48],
                               jnp.zeros((1, HID), _f32)], axis=0)
        wvf = jnp.concatenate([Wv[nb:nb + 16], Wv[nb + 48:],
                               p['emb_edge'] @ Wv[nb + 16:nb + 48],
                               jnp.zeros((1, d_out), _f32)], axis=0)
        wq = _pad_cols(_pad_rows(lp['Wq'], nb + pad), 128)
        wkn = _pad_cols(_pad_rows(Wk[:nb], nb + pad), 128)
        wvn = _pad_cols(_pad_rows(Wv[:nb], nb + pad), 128)
        wkft = _pad_rows(wkf.T, 128)

        drow, srow2 = _pre(h, na, wq, wkn, wvn, wkft, d_in, pad)
        acc3 = _pass12(src, dst, drow, srow2, ef)

        has_ff = i < 5
        if has_ff:
            fp = p['ff'][i]
            w1 = _pad_rows(fp['W1'], d_out + 64 + ((-(d_out + 64)) % 16))
            w2, fg, fb = fp['W2'], fp['ln_g'][None], fp['ln_b'][None]
        else:
            w1 = jnp.zeros((d_out + 64 + ((-(d_out + 64)) % 16), 240), _f32)
            w2 = jnp.zeros((240, 120), _f32)
            fg = jnp.zeros((1, 120), _f32)
            fb = jnp.zeros((1, 120), _f32)
        h = _post(acc3, h, na, wvf, lp['Wo'], lp['ln_g'][None], lp['ln_b'][None],
                  w1, w2, fg, fb, d_in, d_out,
                  residual=(d_out == d_in), has_ff=has_ff,
                  pad=(-(d_out + 64)) % 16)
        d_in = d_out

    o = _head(h, p['out_W1'], p['out_b1'][None],
              _pad_cols(p['out_W2'], 128), _pad_cols(p['out_b2'][None], 128))
    return o[:N, 0:1]
